# initial kernel scaffold (unmeasured)
import jax
import jax.numpy as jnp
from jax import lax
from jax.experimental import pallas as pl
from jax.experimental.pallas import tpu as pltpu


def kernel(
    x,
):
    def body(*refs):
        pass

    out_shape = jax.ShapeDtypeStruct(..., jnp.float32)
    return pl.pallas_call(body, out_shape=out_shape)(...)


# baseline (device time: 1599727 ns/iter reference)
import jax
import jax.numpy as jnp
from jax import lax
from jax.experimental import pallas as pl
from jax.experimental.pallas import tpu as pltpu

CHUNK = 2048
HALF = CHUNK // 2


def kernel(x):
    m, n = x.shape
    n_chunks = m // CHUNK

    def body(x_ref, out_ref, vrecv, hrecv, vs_sem, vr_sem, hs_sem, hr_sem):
        i = pl.program_id(0)
        p = lax.rem(i, 2)
        my_x = lax.axis_index("x")
        my_y = lax.axis_index("y")

        @pl.when(i == 0)
        def _barrier():
            barrier_sem = pltpu.get_barrier_semaphore()
            pl.semaphore_signal(
                barrier_sem, inc=1,
                device_id=(my_x, 1 - my_y),
                device_id_type=pl.DeviceIdType.MESH,
            )
            pl.semaphore_signal(
                barrier_sem, inc=1,
                device_id=(1 - my_x, my_y),
                device_id_type=pl.DeviceIdType.MESH,
            )
            pl.semaphore_wait(barrier_sem, 2)

        half_start = my_x * HALF

        v = pltpu.make_async_remote_copy(
            src_ref=x_ref.at[pl.ds(half_start, HALF), :],
            dst_ref=vrecv.at[p],
            send_sem=vs_sem.at[p],
            recv_sem=vr_sem.at[p],
            device_id=(my_x, 1 - my_y),
            device_id_type=pl.DeviceIdType.MESH,
        )
        v.start()
        v.wait()
        out_ref[pl.ds(half_start, HALF), :] = (
            x_ref[pl.ds(half_start, HALF), :] + vrecv[p]
        )

        h = pltpu.make_async_remote_copy(
            src_ref=out_ref.at[pl.ds(half_start, HALF), :],
            dst_ref=hrecv.at[p],
            send_sem=hs_sem.at[p],
            recv_sem=hr_sem.at[p],
            device_id=(1 - my_x, my_y),
            device_id_type=pl.DeviceIdType.MESH,
        )
        h.start()
        h.wait()
        out_ref[pl.ds((1 - my_x) * HALF, HALF), :] = hrecv[p]

    return pl.pallas_call(
        body,
        grid=(n_chunks,),
        in_specs=[
            pl.BlockSpec((CHUNK, n), lambda i: (i, 0), memory_space=pltpu.VMEM)
        ],
        out_specs=pl.BlockSpec(
            (CHUNK, n), lambda i: (i, 0), memory_space=pltpu.VMEM
        ),
        out_shape=jax.ShapeDtypeStruct((m, n), x.dtype),
        scratch_shapes=[
            pltpu.VMEM((2, HALF, n), x.dtype),
            pltpu.VMEM((2, HALF, n), x.dtype),
            pltpu.SemaphoreType.DMA((2,)),
            pltpu.SemaphoreType.DMA((2,)),
            pltpu.SemaphoreType.DMA((2,)),
            pltpu.SemaphoreType.DMA((2,)),
        ],
        compiler_params=pltpu.CompilerParams(
            collective_id=0,
            dimension_semantics=("arbitrary",),
            vmem_limit_bytes=96 * 1024 * 1024,
        ),
    )(x)


# device time: 904218 ns/iter; 1.7692x vs baseline; 1.7692x over previous
import jax
import jax.numpy as jnp
from jax import lax
from jax.experimental import pallas as pl
from jax.experimental.pallas import tpu as pltpu

CHUNK = 2048
HALF = CHUNK // 2


def kernel(x):
    m, n = x.shape
    n_chunks = m // CHUNK

    def body(x_ref, out_ref, vrecv, hrecv, sbuf,
             vs_sem, vr_sem, hs_sem, hr_sem):
        i = pl.program_id(0)
        my_x = lax.axis_index("x")
        my_y = lax.axis_index("y")
        half_start = my_x * HALF

        def v_copy(slot):
            return pltpu.make_async_remote_copy(
                src_ref=x_ref.at[pl.ds(half_start, HALF), :],
                dst_ref=vrecv.at[slot],
                send_sem=vs_sem.at[slot],
                recv_sem=vr_sem.at[slot],
                device_id=(my_x, 1 - my_y),
                device_id_type=pl.DeviceIdType.MESH,
            )

        def h_copy(slot):
            return pltpu.make_async_remote_copy(
                src_ref=sbuf.at[slot],
                dst_ref=hrecv.at[slot],
                send_sem=hs_sem.at[slot],
                recv_sem=hr_sem.at[slot],
                device_id=(1 - my_x, my_y),
                device_id_type=pl.DeviceIdType.MESH,
            )

        @pl.when(i == 0)
        def _barrier():
            barrier_sem = pltpu.get_barrier_semaphore()
            pl.semaphore_signal(
                barrier_sem, inc=1,
                device_id=(my_x, 1 - my_y),
                device_id_type=pl.DeviceIdType.MESH,
            )
            pl.semaphore_signal(
                barrier_sem, inc=1,
                device_id=(1 - my_x, my_y),
                device_id_type=pl.DeviceIdType.MESH,
            )
            pl.semaphore_wait(barrier_sem, 2)

        p = lax.rem(i, 2)

        @pl.when(i < n_chunks)
        def _v_start():
            v_copy(p).start()

        @pl.when(i > 0)
        def _drain():
            q = lax.rem(i - 1, 2)
            h_copy(q).wait()
            out_ref[pl.ds(half_start, HALF), :] = sbuf[q]
            out_ref[pl.ds((1 - my_x) * HALF, HALF), :] = hrecv[q]

        @pl.when(i < n_chunks)
        def _reduce_and_h():
            v_copy(p).wait()
            sbuf[p] = x_ref[pl.ds(half_start, HALF), :] + vrecv[p]
            h_copy(p).start()

    return pl.pallas_call(
        body,
        grid=(n_chunks + 1,),
        in_specs=[
            pl.BlockSpec(
                (CHUNK, n),
                lambda i: (jnp.minimum(i, n_chunks - 1), 0),
                memory_space=pltpu.VMEM,
            )
        ],
        out_specs=pl.BlockSpec(
            (CHUNK, n),
            lambda i: (jnp.maximum(i - 1, 0), 0),
            memory_space=pltpu.VMEM,
        ),
        out_shape=jax.ShapeDtypeStruct((m, n), x.dtype),
        scratch_shapes=[
            pltpu.VMEM((2, HALF, n), x.dtype),
            pltpu.VMEM((2, HALF, n), x.dtype),
            pltpu.VMEM((2, HALF, n), x.dtype),
            pltpu.SemaphoreType.DMA((2,)),
            pltpu.SemaphoreType.DMA((2,)),
            pltpu.SemaphoreType.DMA((2,)),
            pltpu.SemaphoreType.DMA((2,)),
        ],
        compiler_params=pltpu.CompilerParams(
            collective_id=0,
            dimension_semantics=("arbitrary",),
            vmem_limit_bytes=96 * 1024 * 1024,
        ),
    )(x)


# device time: 865453 ns/iter; 1.8484x vs baseline; 1.0448x over previous
import jax
import jax.numpy as jnp
from jax import lax
from jax.experimental import pallas as pl
from jax.experimental.pallas import tpu as pltpu

CHUNK = 2048
HALF = CHUNK // 2
SUBS = 4
SUB = HALF // SUBS


def kernel(x):
    m, n = x.shape
    n_chunks = m // CHUNK

    def body(x_ref, out_ref, vrecv, hrecv, sbuf,
             vs_sem, vr_sem, hs_sem, hr_sem):
        i = pl.program_id(0)
        my_x = lax.axis_index("x")
        my_y = lax.axis_index("y")
        half_start = my_x * HALF

        def v_copy(slot, s):
            return pltpu.make_async_remote_copy(
                src_ref=x_ref.at[pl.ds(half_start + s * SUB, SUB), :],
                dst_ref=vrecv.at[slot, s],
                send_sem=vs_sem.at[slot, s],
                recv_sem=vr_sem.at[slot, s],
                device_id=(my_x, 1 - my_y),
                device_id_type=pl.DeviceIdType.MESH,
            )

        def h_copy(slot, s):
            return pltpu.make_async_remote_copy(
                src_ref=sbuf.at[slot, s],
                dst_ref=hrecv.at[slot, s],
                send_sem=hs_sem.at[slot, s],
                recv_sem=hr_sem.at[slot, s],
                device_id=(1 - my_x, my_y),
                device_id_type=pl.DeviceIdType.MESH,
            )

        @pl.when(i == 0)
        def _barrier():
            barrier_sem = pltpu.get_barrier_semaphore()
            pl.semaphore_signal(
                barrier_sem, inc=1,
                device_id=(my_x, 1 - my_y),
                device_id_type=pl.DeviceIdType.MESH,
            )
            pl.semaphore_signal(
                barrier_sem, inc=1,
                device_id=(1 - my_x, my_y),
                device_id_type=pl.DeviceIdType.MESH,
            )
            pl.semaphore_wait(barrier_sem, 2)

        p = lax.rem(i, 2)

        @pl.when(i < n_chunks)
        def _v_start():
            for s in range(SUBS):
                v_copy(p, s).start()

        @pl.when(i > 0)
        def _drain():
            q = lax.rem(i - 1, 2)
            for s in range(SUBS):
                h_copy(q, s).wait()
            out_ref[pl.ds(half_start, HALF), :] = sbuf[q].reshape(HALF, n)
            out_ref[pl.ds((1 - my_x) * HALF, HALF), :] = (
                hrecv[q].reshape(HALF, n)
            )

        @pl.when(i < n_chunks)
        def _reduce_and_h():
            for s in range(SUBS):
                v_copy(p, s).wait()
                sbuf[p, s] = (
                    x_ref[pl.ds(half_start + s * SUB, SUB), :] + vrecv[p, s]
                )
                h_copy(p, s).start()

    return pl.pallas_call(
        body,
        grid=(n_chunks + 1,),
        in_specs=[
            pl.BlockSpec(
                (CHUNK, n),
                lambda i: (jnp.minimum(i, n_chunks - 1), 0),
                memory_space=pltpu.VMEM,
            )
        ],
        out_specs=pl.BlockSpec(
            (CHUNK, n),
            lambda i: (jnp.maximum(i - 1, 0), 0),
            memory_space=pltpu.VMEM,
        ),
        out_shape=jax.ShapeDtypeStruct((m, n), x.dtype),
        scratch_shapes=[
            pltpu.VMEM((2, SUBS, SUB, n), x.dtype),
            pltpu.VMEM((2, SUBS, SUB, n), x.dtype),
            pltpu.VMEM((2, SUBS, SUB, n), x.dtype),
            pltpu.SemaphoreType.DMA((2, SUBS)),
            pltpu.SemaphoreType.DMA((2, SUBS)),
            pltpu.SemaphoreType.DMA((2, SUBS)),
            pltpu.SemaphoreType.DMA((2, SUBS)),
        ],
        compiler_params=pltpu.CompilerParams(
            collective_id=0,
            dimension_semantics=("arbitrary",),
            vmem_limit_bytes=96 * 1024 * 1024,
        ),
    )(x)
